# Initial kernel scaffold; baseline (speedup 1.0000x reference)
import jax
import jax.numpy as jnp
from jax.experimental import pallas as pl


def kernel(positions, input, weights1, biases1, weights2, biases2):
    raise NotImplementedError("write your pallas kernel here")


# speedup vs baseline: 1.1807x; 1.1807x over previous
import functools

import numpy as np
import jax
import jax.numpy as jnp
from jax.experimental import pallas as pl
from jax.experimental.pallas import tpu as pltpu

_CUTOFF = 5.0
_NG = 32
_WIDTH = _CUTOFF / (_NG - 1)
_LOG2 = np.float32(np.log(2.0))


def _ssp(x):
    return jnp.maximum(x, 0.0) + jnp.log1p(jnp.exp(-jnp.abs(x))) - _LOG2


def _dense_kernel(px_ref, py_ref, pz_ref, x_ref, w1_ref, b1_ref, w2_ref,
                  b2_ref, out_ref, *, bd, bs):
    d = pl.program_id(0)
    s = pl.program_id(1)
    sx = x_ref[...]
    w1 = w1_ref[...]
    w2 = w2_ref[...]
    b1 = b1_ref[...]
    b2 = b2_ref[...]
    sxp = px_ref[pl.ds(s * bs, bs), :]
    syp = py_ref[pl.ds(s * bs, bs), :]
    szp = pz_ref[pl.ds(s * bs, bs), :]
    row_ids = jax.lax.broadcasted_iota(jnp.int32, (bs, 1), 0) + s * bs
    centers = (jax.lax.broadcasted_iota(jnp.int32, (1, _NG), 1)
               .astype(jnp.float32) * np.float32(_WIDTH))

    @pl.when(s == 0)
    def _zero():
        out_ref[...] = jnp.zeros_like(out_ref)

    def body(li, _):
        ig = d * bd + li
        pd_x = px_ref[pl.ds(ig, 1), :]
        pd_y = py_ref[pl.ds(ig, 1), :]
        pd_z = pz_ref[pl.ds(ig, 1), :]
        dx = sxp - pd_x
        dy = syp - pd_y
        dz = szp - pd_z
        dist2 = dx * dx + dy * dy + dz * dz
        mask = (dist2 < np.float32(_CUTOFF * _CUTOFF)) & (row_ids != ig)
        r = jnp.sqrt(dist2 + np.float32(1e-12))
        t = (r - centers) * np.float32(1.0 / _WIDTH)
        rbf = jnp.exp(np.float32(-0.5) * t * t)
        y = _ssp(jnp.dot(rbf, w1, preferred_element_type=jnp.float32) + b1)
        w = _ssp(jnp.dot(y, w2, preferred_element_type=jnp.float32) + b2)
        cut = np.float32(0.5) * jnp.cos(np.float32(np.pi / _CUTOFF) * r) \
            + np.float32(0.5)
        w = jnp.where(mask, w * cut, np.float32(0.0))
        out_ref[pl.ds(li, 1), :] += jnp.sum(w * sx, axis=0, keepdims=True)
        return 0

    jax.lax.fori_loop(0, bd, body, 0)


def _cfconv_dense(px, py, pz, x, w1, b1, w2, b2):
    n, nf = x.shape
    bd, bs = 64, 1024
    grid = (n // bd, n // bs)
    body = functools.partial(_dense_kernel, bd=bd, bs=bs)
    return pl.pallas_call(
        body,
        grid=grid,
        in_specs=[
            pl.BlockSpec((n, 1), lambda d, s: (0, 0)),
            pl.BlockSpec((n, 1), lambda d, s: (0, 0)),
            pl.BlockSpec((n, 1), lambda d, s: (0, 0)),
            pl.BlockSpec((bs, nf), lambda d, s: (s, 0)),
            pl.BlockSpec(w1.shape, lambda d, s: (0, 0)),
            pl.BlockSpec(b1.shape, lambda d, s: (0, 0)),
            pl.BlockSpec(w2.shape, lambda d, s: (0, 0)),
            pl.BlockSpec(b2.shape, lambda d, s: (0, 0)),
        ],
        out_specs=pl.BlockSpec((bd, nf), lambda d, s: (d, 0)),
        out_shape=jax.ShapeDtypeStruct((n, nf), jnp.float32),
        compiler_params=pltpu.CompilerParams(
            dimension_semantics=("parallel", "arbitrary"),
        ),
    )(px, py, pz, x, w1, b1, w2, b2)


def kernel(positions, input, weights1, biases1, weights2, biases2):
    px = positions[:, 0:1]
    py = positions[:, 1:2]
    pz = positions[:, 2:3]
    b1 = biases1.reshape(1, -1)
    b2 = biases2.reshape(1, -1)
    return _cfconv_dense(px, py, pz, input, weights1, b1, weights2, b2)


# speedup vs baseline: 74.5771x; 63.1640x over previous
import functools

import numpy as np
import jax
import jax.numpy as jnp
from jax import lax
from jax.experimental import pallas as pl
from jax.experimental.pallas import tpu as pltpu
from jax.experimental.pallas import tpu_sc as plsc

_CUTOFF = 5.0
_NG = 32
_WIDTH = _CUTOFF / (_NG - 1)
_LOG2 = np.float32(np.log(2.0))
_T = 65536
_D2SCALE = np.float32(_T / (_CUTOFF * _CUTOFF))
_KCAP = 256


def _ssp(x):
    return jnp.maximum(x, 0.0) + jnp.log1p(jnp.exp(-jnp.abs(x))) - _LOG2


def _tab_kernel(w1_ref, b1_ref, w2_ref, b2_ref, out_ref, *, bt):
    p = pl.program_id(0)
    kk = jax.lax.broadcasted_iota(jnp.int32, (bt, 1), 0) + p * bt
    d2 = (kk.astype(jnp.float32) + np.float32(0.5)) \
        * np.float32(_CUTOFF * _CUTOFF / _T)
    r = jnp.sqrt(d2 + np.float32(1e-12))
    centers = (jax.lax.broadcasted_iota(jnp.int32, (1, _NG), 1)
               .astype(jnp.float32) * np.float32(_WIDTH))
    t = (r - centers) * np.float32(1.0 / _WIDTH)
    rbf = jnp.exp(np.float32(-0.5) * t * t)
    y = _ssp(jnp.dot(rbf, w1_ref[...], preferred_element_type=jnp.float32)
             + b1_ref[...])
    w = _ssp(jnp.dot(y, w2_ref[...], preferred_element_type=jnp.float32)
             + b2_ref[...])
    cut = np.float32(0.5) * jnp.cos(np.float32(np.pi / _CUTOFF) * r) \
        + np.float32(0.5)
    out_ref[...] = w * cut


def _build_table(w1, b1, w2, b2):
    nf = w1.shape[1]
    bt = 4096
    return pl.pallas_call(
        functools.partial(_tab_kernel, bt=bt),
        grid=(_T // bt,),
        in_specs=[
            pl.BlockSpec(w1.shape, lambda p: (0, 0)),
            pl.BlockSpec(b1.shape, lambda p: (0, 0)),
            pl.BlockSpec(w2.shape, lambda p: (0, 0)),
            pl.BlockSpec(b2.shape, lambda p: (0, 0)),
        ],
        out_specs=pl.BlockSpec((bt, nf), lambda p: (p, 0)),
        out_shape=jax.ShapeDtypeStruct((_T, nf), jnp.float32),
        compiler_params=pltpu.CompilerParams(
            dimension_semantics=("parallel",),
        ),
    )(w1, b1, w2, b2)


def _sc_conv(px, py, pz, x, wtab):
    n, nf = x.shape
    nc, ns = 2, 16
    nw = nc * ns
    na = n // nw
    mesh = plsc.VectorSubcoreMesh(core_axis_name="c", subcore_axis_name="s")

    @functools.partial(
        pl.kernel, mesh=mesh,
        out_type=jax.ShapeDtypeStruct((n, nf), jnp.float32),
        compiler_params=pltpu.CompilerParams(
            needs_layout_passes=False, use_tc_tiling_on_sc=False),
        scratch_types=[
            pltpu.VMEM((n,), jnp.float32),
            pltpu.VMEM((n,), jnp.float32),
            pltpu.VMEM((n,), jnp.float32),
            pltpu.VMEM((_KCAP,), jnp.int32),
            pltpu.VMEM((_KCAP,), jnp.float32),
            pltpu.VMEM((16, nf), jnp.float32),
            pltpu.VMEM((16, nf), jnp.float32),
            pltpu.VMEM((na, nf), jnp.float32),
            pltpu.SemaphoreType.DMA,
            pltpu.SemaphoreType.DMA,
        ],
    )
    def conv(px_hbm, py_hbm, pz_hbm, x_hbm, wtab_hbm, out_hbm,
             pxv, pyv, pzv, nidx, nd2, wrow, xrow, obuf, sem0, sem1):
        wid = lax.axis_index("s") * nc + lax.axis_index("c")
        base = wid * na
        pltpu.sync_copy(px_hbm, pxv)
        pltpu.sync_copy(py_hbm, pyv)
        pltpu.sync_copy(pz_hbm, pzv)

        lane = lax.iota(jnp.int32, 16)
        zero = jnp.zeros((16,), jnp.float32)

        def atom_body(i_loc, _):
            ig = base + i_loc
            igv = jnp.full((16,), ig, jnp.int32)
            cb16 = (ig // 16) * 16
            lsel = igv - cb16
            pix = pxv[pl.ds(cb16, 16)].at[lsel].get(
                mode="promise_in_bounds")
            piy = pyv[pl.ds(cb16, 16)].at[lsel].get(
                mode="promise_in_bounds")
            piz = pzv[pl.ds(cb16, 16)].at[lsel].get(
                mode="promise_in_bounds")

            def clr(cc, _):
                nd2[pl.ds(cc * 16, 16)] = jnp.full(
                    (16,), np.float32(24.999), jnp.float32)
                return 0

            lax.fori_loop(0, _KCAP // 16, clr, 0)

            def scan_body(b, cnt):
                off = b * 16
                dx = pxv[pl.ds(off, 16)] - pix
                dy = pyv[pl.ds(off, 16)] - piy
                dz = pzv[pl.ds(off, 16)] - piz
                d2 = dx * dx + dy * dy + dz * dz
                src = lane + off
                m = (d2 < np.float32(_CUTOFF * _CUTOFF)) & (src != igv)
                csum = plsc.cumsum(m.astype(jnp.int32))
                p = cnt + csum - 1
                plsc.store_scatter(nidx, [p], src, mask=m)
                plsc.store_scatter(nd2, [p], d2, mask=m)
                return cnt + jnp.max(csum)

            cnt = lax.fori_loop(0, n // 16, scan_body, 0)
            nch = (cnt + 15) // 16

            def chunk_body(cb, accs):
                off = cb * 16
                d2c = nd2[pl.ds(off, 16)]
                kv = jnp.minimum((d2c * _D2SCALE).astype(jnp.int32), _T - 1)
                iv = nidx[pl.ds(off, 16)] & (n - 1)
                cp0 = pltpu.async_copy(wtab_hbm.at[kv], wrow, sem0)
                cp1 = pltpu.async_copy(x_hbm.at[iv], xrow, sem1)
                cp0.wait()
                cp1.wait()

                def row_body(j, bb):
                    b0, b1, b2, b3 = bb
                    b0 = b0 + wrow[j, pl.ds(0, 16)] * xrow[j, pl.ds(0, 16)]
                    b1 = b1 + wrow[j, pl.ds(16, 16)] * xrow[j, pl.ds(16, 16)]
                    b2 = b2 + wrow[j, pl.ds(32, 16)] * xrow[j, pl.ds(32, 16)]
                    b3 = b3 + wrow[j, pl.ds(48, 16)] * xrow[j, pl.ds(48, 16)]
                    return (b0, b1, b2, b3)

                return lax.fori_loop(0, 16, row_body, accs)

            a0, a1, a2, a3 = lax.fori_loop(0, nch, chunk_body,
                                           (zero, zero, zero, zero))
            obuf[i_loc, pl.ds(0, 16)] = a0
            obuf[i_loc, pl.ds(16, 16)] = a1
            obuf[i_loc, pl.ds(32, 16)] = a2
            obuf[i_loc, pl.ds(48, 16)] = a3
            return 0

        lax.fori_loop(0, na, atom_body, 0)
        pltpu.sync_copy(obuf, out_hbm.at[pl.ds(base, na)])

    return conv(px, py, pz, x, wtab)


def kernel(positions, input, weights1, biases1, weights2, biases2):
    px = positions[:, 0]
    py = positions[:, 1]
    pz = positions[:, 2]
    b1 = biases1.reshape(1, -1)
    b2 = biases2.reshape(1, -1)
    wtab = _build_table(weights1, b1, weights2, b2)
    return _sc_conv(px, py, pz, input, wtab)


# speedup vs baseline: 132.1164x; 1.7715x over previous
import functools

import numpy as np
import jax
import jax.numpy as jnp
from jax import lax
from jax.experimental import pallas as pl
from jax.experimental.pallas import tpu as pltpu
from jax.experimental.pallas import tpu_sc as plsc

_CUTOFF = 5.0
_NG = 32
_WIDTH = _CUTOFF / (_NG - 1)
_LOG2 = np.float32(np.log(2.0))
_T = 65536
_D2SCALE = np.float32(_T / (_CUTOFF * _CUTOFF))
_KCAP = 256
_BOX = 51.2
_NCD = 10
_CINV = np.float32(_NCD / _BOX)
_NCELL_PAD = 1040
_NPAD = 16


def _ssp(x):
    return jnp.maximum(x, 0.0) + jnp.log1p(jnp.exp(-jnp.abs(x))) - _LOG2


def _floor_i32(x):
    k = x.astype(jnp.int32)
    return k - (k.astype(jnp.float32) > x).astype(jnp.int32)


def _tab_kernel(w1_ref, b1_ref, w2_ref, b2_ref, out_ref, *, bt):
    p = pl.program_id(0)
    kk = jax.lax.broadcasted_iota(jnp.int32, (bt, 1), 0) + p * bt
    d2 = (kk.astype(jnp.float32) + np.float32(0.5)) \
        * np.float32(_CUTOFF * _CUTOFF / _T)
    r = jnp.sqrt(d2 + np.float32(1e-12))
    centers = (jax.lax.broadcasted_iota(jnp.int32, (1, _NG), 1)
               .astype(jnp.float32) * np.float32(_WIDTH))
    t = (r - centers) * np.float32(1.0 / _WIDTH)
    rbf = jnp.exp(np.float32(-0.5) * t * t)
    y = _ssp(jnp.dot(rbf, w1_ref[...], preferred_element_type=jnp.float32)
             + b1_ref[...])
    w = _ssp(jnp.dot(y, w2_ref[...], preferred_element_type=jnp.float32)
             + b2_ref[...])
    cut = np.float32(0.5) * jnp.cos(np.float32(np.pi / _CUTOFF) * r) \
        + np.float32(0.5)
    out_ref[...] = w * cut


def _build_table(w1, b1, w2, b2):
    nf = w1.shape[1]
    bt = 4096
    return pl.pallas_call(
        functools.partial(_tab_kernel, bt=bt),
        grid=(_T // bt,),
        in_specs=[
            pl.BlockSpec(w1.shape, lambda p: (0, 0)),
            pl.BlockSpec(b1.shape, lambda p: (0, 0)),
            pl.BlockSpec(w2.shape, lambda p: (0, 0)),
            pl.BlockSpec(b2.shape, lambda p: (0, 0)),
        ],
        out_specs=pl.BlockSpec((bt, nf), lambda p: (p, 0)),
        out_shape=jax.ShapeDtypeStruct((_T, nf), jnp.float32),
        compiler_params=pltpu.CompilerParams(
            dimension_semantics=("parallel",),
        ),
    )(w1, b1, w2, b2)


def _sc_conv(px, py, pz, x, wtab):
    n, nf = x.shape
    nc, ns = 2, 16
    nw = nc * ns
    na = n // nw
    npad = n + _NPAD
    mesh = plsc.VectorSubcoreMesh(core_axis_name="c", subcore_axis_name="s")

    @functools.partial(
        pl.kernel, mesh=mesh,
        out_type=jax.ShapeDtypeStruct((n, nf), jnp.float32),
        compiler_params=pltpu.CompilerParams(
            needs_layout_passes=False, use_tc_tiling_on_sc=False),
        scratch_types=[
            pltpu.VMEM((npad,), jnp.float32),
            pltpu.VMEM((npad,), jnp.float32),
            pltpu.VMEM((npad,), jnp.float32),
            pltpu.VMEM((npad,), jnp.int32),
            pltpu.SMEM((1024,), jnp.int32),
            pltpu.VMEM((npad,), jnp.float32),
            pltpu.VMEM((npad,), jnp.float32),
            pltpu.VMEM((npad,), jnp.float32),
            pltpu.VMEM((npad,), jnp.int32),
            pltpu.VMEM((_KCAP,), jnp.int32),
            pltpu.VMEM((_KCAP,), jnp.float32),
            pltpu.VMEM((16, nf), jnp.float32),
            pltpu.VMEM((16, nf), jnp.float32),
            pltpu.VMEM((na, nf), jnp.float32),
            pltpu.SemaphoreType.DMA,
            pltpu.SemaphoreType.DMA,
        ],
    )
    def conv(px_hbm, py_hbm, pz_hbm, x_hbm, wtab_hbm, out_hbm,
             pxv, pyv, pzv, cid, cellptr,
             spx, spy, spz, sidx, nidx, nd2, wrow, xrow, obuf, sem0, sem1):
        wid = lax.axis_index("s") * nc + lax.axis_index("c")
        base = wid * na
        pltpu.sync_copy(px_hbm, pxv.at[pl.ds(0, n)])
        pltpu.sync_copy(py_hbm, pyv.at[pl.ds(0, n)])
        pltpu.sync_copy(pz_hbm, pzv.at[pl.ds(0, n)])

        lane = lax.iota(jnp.int32, 16)
        l0 = lane == 0
        zero = jnp.zeros((16,), jnp.float32)
        izero = jnp.zeros((16,), jnp.int32)

        def cid_body(b, _):
            off = b * 16
            cx = _floor_i32(pxv[pl.ds(off, 16)] * _CINV)
            cy = _floor_i32(pyv[pl.ds(off, 16)] * _CINV)
            cz = _floor_i32(pzv[pl.ds(off, 16)] * _CINV)
            cid[pl.ds(off, 16)] = (cx * _NCD + cy) * _NCD + cz
            return 0

        lax.fori_loop(0, n // 16, cid_body, 0)

        def clr_counts(i, _):
            cellptr[i] = 0
            return 0

        lax.fori_loop(0, 1024, clr_counts, 0)

        def count_body(i, _):
            c = cid[pl.ds(i, 16)][0]
            cellptr[c] = cellptr[c] + 1
            return 0

        lax.fori_loop(0, n, count_body, 0)

        def pfx_body(c, tot):
            v = cellptr[c]
            cellptr[c] = tot
            return tot + v

        lax.fori_loop(0, 1000, pfx_body, 0)

        def place_body(i, _):
            c = cid[pl.ds(i, 16)][0]
            p = cellptr[c]
            cellptr[c] = p + 1
            pv = jnp.full((16,), p, jnp.int32)
            plsc.store_scatter(sidx, [pv], jnp.full((16,), i, jnp.int32),
                               mask=l0)
            vx = pxv[pl.ds(i, 16)][0]
            vy = pyv[pl.ds(i, 16)][0]
            vz = pzv[pl.ds(i, 16)][0]
            plsc.store_scatter(spx, [pv], jnp.full((16,), vx, jnp.float32),
                               mask=l0)
            plsc.store_scatter(spy, [pv], jnp.full((16,), vy, jnp.float32),
                               mask=l0)
            plsc.store_scatter(spz, [pv], jnp.full((16,), vz, jnp.float32),
                               mask=l0)
            return 0

        lax.fori_loop(0, n, place_body, 0)

        far = jnp.full((16,), np.float32(1e9), jnp.float32)
        spx[pl.ds(n, 16)] = far
        spy[pl.ds(n, 16)] = far
        spz[pl.ds(n, 16)] = far
        sidx[pl.ds(n, 16)] = izero

        def atom_body(i_loc, _):
            ig = base + i_loc
            igv = jnp.full((16,), ig, jnp.int32)
            cb16 = (ig // 16) * 16
            lsel = igv - cb16
            pix = pxv[pl.ds(cb16, 16)].at[lsel].get(mode="promise_in_bounds")
            piy = pyv[pl.ds(cb16, 16)].at[lsel].get(mode="promise_in_bounds")
            piz = pzv[pl.ds(cb16, 16)].at[lsel].get(mode="promise_in_bounds")
            cx = _floor_i32(pix * _CINV)[0]
            cy = _floor_i32(piy * _CINV)[0]
            cz = _floor_i32(piz * _CINV)[0]

            def clr(cc, _):
                nd2[pl.ds(cc * 16, 16)] = jnp.full(
                    (16,), np.float32(24.999), jnp.float32)
                return 0

            lax.fori_loop(0, _KCAP // 16, clr, 0)

            zlo = jnp.maximum(cz - 1, 0)
            zhi = jnp.minimum(cz + 1, _NCD - 1)

            def col_body(q, cnt):
                ncx = cx + q // 3 - 1
                ncy = cy + q % 3 - 1
                valid = ((ncx >= 0) & (ncx < _NCD)
                         & (ncy >= 0) & (ncy < _NCD))
                cbase = (ncx * _NCD + ncy) * _NCD
                ilo = cbase + zlo - 1
                s = jnp.where(ilo < 0, 0, cellptr[jnp.maximum(ilo, 0)])
                e = cellptr[jnp.clip(cbase + zhi, 0, 1023)]
                e = jnp.where(valid, e, s)
                m_count = e - s
                nchk = (m_count + 15) // 16

                def cand_body(qq, cnt2):
                    off = s + qq * 16
                    jx = spx[pl.ds(off, 16)]
                    jy = spy[pl.ds(off, 16)]
                    jz = spz[pl.ds(off, 16)]
                    jidx = sidx[pl.ds(off, 16)]
                    rem = m_count - qq * 16
                    dx = jx - pix
                    dy = jy - piy
                    dz = jz - piz
                    d2 = dx * dx + dy * dy + dz * dz
                    m = ((lane < rem)
                         & (d2 < np.float32(_CUTOFF * _CUTOFF))
                         & (jidx != igv))
                    csum = plsc.cumsum(m.astype(jnp.int32))
                    p = cnt2 + csum - 1
                    plsc.store_scatter(nidx, [p], jidx, mask=m)
                    plsc.store_scatter(nd2, [p], d2, mask=m)
                    return cnt2 + jnp.max(csum)

                return lax.fori_loop(0, nchk, cand_body, cnt)

            cnt = lax.fori_loop(0, 9, col_body, 0)
            nch = (cnt + 15) // 16

            def chunk_body(cb, accs):
                off = cb * 16
                d2c = nd2[pl.ds(off, 16)]
                kv = jnp.minimum((d2c * _D2SCALE).astype(jnp.int32), _T - 1)
                iv = nidx[pl.ds(off, 16)] & (n - 1)
                cp0 = pltpu.async_copy(wtab_hbm.at[kv], wrow, sem0)
                cp1 = pltpu.async_copy(x_hbm.at[iv], xrow, sem1)
                cp0.wait()
                cp1.wait()

                def row_body(j, bb):
                    b0, b1, b2, b3 = bb
                    b0 = b0 + wrow[j, pl.ds(0, 16)] * xrow[j, pl.ds(0, 16)]
                    b1 = b1 + wrow[j, pl.ds(16, 16)] * xrow[j, pl.ds(16, 16)]
                    b2 = b2 + wrow[j, pl.ds(32, 16)] * xrow[j, pl.ds(32, 16)]
                    b3 = b3 + wrow[j, pl.ds(48, 16)] * xrow[j, pl.ds(48, 16)]
                    return (b0, b1, b2, b3)

                return lax.fori_loop(0, 16, row_body, accs)

            a0, a1, a2, a3 = lax.fori_loop(0, nch, chunk_body,
                                           (zero, zero, zero, zero))
            obuf[i_loc, pl.ds(0, 16)] = a0
            obuf[i_loc, pl.ds(16, 16)] = a1
            obuf[i_loc, pl.ds(32, 16)] = a2
            obuf[i_loc, pl.ds(48, 16)] = a3
            return 0

        lax.fori_loop(0, na, atom_body, 0)
        pltpu.sync_copy(obuf, out_hbm.at[pl.ds(base, na)])

    return conv(px, py, pz, x, wtab)


def kernel(positions, input, weights1, biases1, weights2, biases2):
    px = positions[:, 0]
    py = positions[:, 1]
    pz = positions[:, 2]
    b1 = biases1.reshape(1, -1)
    b2 = biases2.reshape(1, -1)
    wtab = _build_table(weights1, b1, weights2, b2)
    return _sc_conv(px, py, pz, input, wtab)


# speedup vs baseline: 132.7864x; 1.0051x over previous
import functools

import numpy as np
import jax
import jax.numpy as jnp
from jax import lax
from jax.experimental import pallas as pl
from jax.experimental.pallas import tpu as pltpu
from jax.experimental.pallas import tpu_sc as plsc

_CUTOFF = 5.0
_NG = 32
_WIDTH = _CUTOFF / (_NG - 1)
_LOG2 = np.float32(np.log(2.0))
_T = 65536
_D2SCALE = np.float32(_T / (_CUTOFF * _CUTOFF))
_KCAP = 256
_BOX = 51.2
_NCD = 10
_CINV = np.float32(_NCD / _BOX)
_NCELL_PAD = 1040
_NPAD = 16


def _ssp(x):
    return jnp.maximum(x, 0.0) + jnp.log1p(jnp.exp(-jnp.abs(x))) - _LOG2


def _floor_i32(x):
    k = x.astype(jnp.int32)
    return k - (k.astype(jnp.float32) > x).astype(jnp.int32)


def _tab_kernel(w1_ref, b1_ref, w2_ref, b2_ref, out_ref, *, bt):
    p = pl.program_id(0)
    kk = jax.lax.broadcasted_iota(jnp.int32, (bt, 1), 0) + p * bt
    d2 = (kk.astype(jnp.float32) + np.float32(0.5)) \
        * np.float32(_CUTOFF * _CUTOFF / _T)
    r = jnp.sqrt(d2 + np.float32(1e-12))
    centers = (jax.lax.broadcasted_iota(jnp.int32, (1, _NG), 1)
               .astype(jnp.float32) * np.float32(_WIDTH))
    t = (r - centers) * np.float32(1.0 / _WIDTH)
    rbf = jnp.exp(np.float32(-0.5) * t * t)
    y = _ssp(jnp.dot(rbf, w1_ref[...], preferred_element_type=jnp.float32)
             + b1_ref[...])
    w = _ssp(jnp.dot(y, w2_ref[...], preferred_element_type=jnp.float32)
             + b2_ref[...])
    cut = np.float32(0.5) * jnp.cos(np.float32(np.pi / _CUTOFF) * r) \
        + np.float32(0.5)
    out_ref[...] = w * cut


def _build_table(w1, b1, w2, b2):
    nf = w1.shape[1]
    bt = 4096
    return pl.pallas_call(
        functools.partial(_tab_kernel, bt=bt),
        grid=(_T // bt,),
        in_specs=[
            pl.BlockSpec(w1.shape, lambda p: (0, 0)),
            pl.BlockSpec(b1.shape, lambda p: (0, 0)),
            pl.BlockSpec(w2.shape, lambda p: (0, 0)),
            pl.BlockSpec(b2.shape, lambda p: (0, 0)),
        ],
        out_specs=pl.BlockSpec((bt, nf), lambda p: (p, 0)),
        out_shape=jax.ShapeDtypeStruct((_T, nf), jnp.float32),
        compiler_params=pltpu.CompilerParams(
            dimension_semantics=("parallel",),
        ),
    )(w1, b1, w2, b2)


def _sc_conv(px, py, pz, x, wtab):
    n, nf = x.shape
    nc, ns = 2, 16
    nw = nc * ns
    na = n // nw
    npad = n + _NPAD
    mesh = plsc.VectorSubcoreMesh(core_axis_name="c", subcore_axis_name="s")

    @functools.partial(
        pl.kernel, mesh=mesh,
        out_type=jax.ShapeDtypeStruct((n, nf), jnp.float32),
        compiler_params=pltpu.CompilerParams(
            needs_layout_passes=False, use_tc_tiling_on_sc=False),
        scratch_types=[
            pltpu.VMEM((npad,), jnp.float32),
            pltpu.VMEM((npad,), jnp.float32),
            pltpu.VMEM((npad,), jnp.float32),
            pltpu.VMEM((npad,), jnp.int32),
            pltpu.SMEM((1024,), jnp.int32),
            pltpu.VMEM((npad,), jnp.float32),
            pltpu.VMEM((npad,), jnp.float32),
            pltpu.VMEM((npad,), jnp.float32),
            pltpu.VMEM((npad,), jnp.int32),
            pltpu.VMEM((_KCAP,), jnp.int32),
            pltpu.VMEM((_KCAP,), jnp.float32),
            pltpu.VMEM((160, nf), jnp.float32),
            pltpu.VMEM((160, nf), jnp.float32),
            pltpu.VMEM((na, nf), jnp.float32),
            pltpu.SemaphoreType.DMA,
            pltpu.SemaphoreType.DMA,
        ],
    )
    def conv(px_hbm, py_hbm, pz_hbm, x_hbm, wtab_hbm, out_hbm,
             pxv, pyv, pzv, cid, cellptr,
             spx, spy, spz, sidx, nidx, nd2, wrow, xrow, obuf, sem0, sem1):
        wid = lax.axis_index("s") * nc + lax.axis_index("c")
        base = wid * na
        pltpu.sync_copy(px_hbm, pxv.at[pl.ds(0, n)])
        pltpu.sync_copy(py_hbm, pyv.at[pl.ds(0, n)])
        pltpu.sync_copy(pz_hbm, pzv.at[pl.ds(0, n)])

        lane = lax.iota(jnp.int32, 16)
        l0 = lane == 0
        zero = jnp.zeros((16,), jnp.float32)
        izero = jnp.zeros((16,), jnp.int32)

        def cid_body(b, _):
            off = b * 16
            cx = _floor_i32(pxv[pl.ds(off, 16)] * _CINV)
            cy = _floor_i32(pyv[pl.ds(off, 16)] * _CINV)
            cz = _floor_i32(pzv[pl.ds(off, 16)] * _CINV)
            cid[pl.ds(off, 16)] = (cx * _NCD + cy) * _NCD + cz
            return 0

        lax.fori_loop(0, n // 16, cid_body, 0)

        def clr_counts(i, _):
            cellptr[i] = 0
            return 0

        lax.fori_loop(0, 1024, clr_counts, 0)

        def count_body(i, _):
            c = cid[pl.ds(i, 16)][0]
            cellptr[c] = cellptr[c] + 1
            return 0

        lax.fori_loop(0, n, count_body, 0)

        def pfx_body(c, tot):
            v = cellptr[c]
            cellptr[c] = tot
            return tot + v

        lax.fori_loop(0, 1000, pfx_body, 0)

        def place_body(i, _):
            c = cid[pl.ds(i, 16)][0]
            p = cellptr[c]
            cellptr[c] = p + 1
            pv = jnp.full((16,), p, jnp.int32)
            plsc.store_scatter(sidx, [pv], jnp.full((16,), i, jnp.int32),
                               mask=l0)
            vx = pxv[pl.ds(i, 16)][0]
            vy = pyv[pl.ds(i, 16)][0]
            vz = pzv[pl.ds(i, 16)][0]
            plsc.store_scatter(spx, [pv], jnp.full((16,), vx, jnp.float32),
                               mask=l0)
            plsc.store_scatter(spy, [pv], jnp.full((16,), vy, jnp.float32),
                               mask=l0)
            plsc.store_scatter(spz, [pv], jnp.full((16,), vz, jnp.float32),
                               mask=l0)
            return 0

        lax.fori_loop(0, n, place_body, 0)

        far = jnp.full((16,), np.float32(1e9), jnp.float32)
        spx[pl.ds(n, 16)] = far
        spy[pl.ds(n, 16)] = far
        spz[pl.ds(n, 16)] = far
        sidx[pl.ds(n, 16)] = izero

        def atom_body(i_loc, _):
            ig = base + i_loc
            igv = jnp.full((16,), ig, jnp.int32)
            cb16 = (ig // 16) * 16
            lsel = igv - cb16
            pix = pxv[pl.ds(cb16, 16)].at[lsel].get(mode="promise_in_bounds")
            piy = pyv[pl.ds(cb16, 16)].at[lsel].get(mode="promise_in_bounds")
            piz = pzv[pl.ds(cb16, 16)].at[lsel].get(mode="promise_in_bounds")
            cx = _floor_i32(pix * _CINV)[0]
            cy = _floor_i32(piy * _CINV)[0]
            cz = _floor_i32(piz * _CINV)[0]

            def clr(cc, _):
                nd2[pl.ds(cc * 16, 16)] = jnp.full(
                    (16,), np.float32(24.999), jnp.float32)
                return 0

            lax.fori_loop(0, _KCAP // 16, clr, 0)

            zlo = jnp.maximum(cz - 1, 0)
            zhi = jnp.minimum(cz + 1, _NCD - 1)

            def col_body(q, cnt):
                ncx = cx + q // 3 - 1
                ncy = cy + q % 3 - 1
                valid = ((ncx >= 0) & (ncx < _NCD)
                         & (ncy >= 0) & (ncy < _NCD))
                cbase = (ncx * _NCD + ncy) * _NCD
                ilo = cbase + zlo - 1
                s = jnp.where(ilo < 0, 0, cellptr[jnp.maximum(ilo, 0)])
                e = cellptr[jnp.clip(cbase + zhi, 0, 1023)]
                e = jnp.where(valid, e, s)
                m_count = e - s
                nchk = (m_count + 15) // 16

                def cand_body(qq, cnt2):
                    off = s + qq * 16
                    jx = spx[pl.ds(off, 16)]
                    jy = spy[pl.ds(off, 16)]
                    jz = spz[pl.ds(off, 16)]
                    jidx = sidx[pl.ds(off, 16)]
                    rem = m_count - qq * 16
                    dx = jx - pix
                    dy = jy - piy
                    dz = jz - piz
                    d2 = dx * dx + dy * dy + dz * dz
                    m = ((lane < rem)
                         & (d2 < np.float32(_CUTOFF * _CUTOFF))
                         & (jidx != igv))
                    csum = plsc.cumsum(m.astype(jnp.int32))
                    p = cnt2 + csum - 1
                    plsc.store_scatter(nidx, [p], jidx, mask=m)
                    plsc.store_scatter(nd2, [p], d2, mask=m)
                    return cnt2 + jnp.max(csum)

                return lax.fori_loop(0, nchk, cand_body, cnt)

            cnt = lax.fori_loop(0, 9, col_body, 0)
            nch = jnp.minimum((cnt + 15) // 16, 10)

            def issue_body(cb, _):
                off = cb * 16
                d2c = nd2[pl.ds(off, 16)]
                kv = jnp.minimum((d2c * _D2SCALE).astype(jnp.int32), _T - 1)
                iv = nidx[pl.ds(off, 16)] & (n - 1)
                pltpu.async_copy(wtab_hbm.at[kv],
                                 wrow.at[pl.ds(off, 16), :], sem0)
                pltpu.async_copy(x_hbm.at[iv],
                                 xrow.at[pl.ds(off, 16), :], sem1)
                return 0

            lax.fori_loop(0, nch, issue_body, 0)

            def drain_body(cb, _):
                off = cb * 16
                pltpu.make_async_copy(wtab_hbm.at[izero],
                                      wrow.at[pl.ds(off, 16), :], sem0).wait()
                pltpu.make_async_copy(x_hbm.at[izero],
                                      xrow.at[pl.ds(off, 16), :], sem1).wait()
                return 0

            lax.fori_loop(0, nch, drain_body, 0)

            def row_body(j, bb):
                b0, b1, b2, b3 = bb
                b0 = b0 + wrow[j, pl.ds(0, 16)] * xrow[j, pl.ds(0, 16)]
                b1 = b1 + wrow[j, pl.ds(16, 16)] * xrow[j, pl.ds(16, 16)]
                b2 = b2 + wrow[j, pl.ds(32, 16)] * xrow[j, pl.ds(32, 16)]
                b3 = b3 + wrow[j, pl.ds(48, 16)] * xrow[j, pl.ds(48, 16)]
                return (b0, b1, b2, b3)

            a0, a1, a2, a3 = lax.fori_loop(0, nch * 16, row_body,
                                           (zero, zero, zero, zero))
            obuf[i_loc, pl.ds(0, 16)] = a0
            obuf[i_loc, pl.ds(16, 16)] = a1
            obuf[i_loc, pl.ds(32, 16)] = a2
            obuf[i_loc, pl.ds(48, 16)] = a3
            return 0

        lax.fori_loop(0, na, atom_body, 0)
        pltpu.sync_copy(obuf, out_hbm.at[pl.ds(base, na)])

    return conv(px, py, pz, x, wtab)


def kernel(positions, input, weights1, biases1, weights2, biases2):
    px = positions[:, 0]
    py = positions[:, 1]
    pz = positions[:, 2]
    b1 = biases1.reshape(1, -1)
    b2 = biases2.reshape(1, -1)
    wtab = _build_table(weights1, b1, weights2, b2)
    return _sc_conv(px, py, pz, input, wtab)
